# Initial kernel scaffold; baseline (speedup 1.0000x reference)
import jax
import jax.numpy as jnp
from jax.experimental import pallas as pl


def kernel(logits, dup_mask):
    raise NotImplementedError("write your pallas kernel here")


# speedup vs baseline: 5.0445x; 5.0445x over previous
import functools

import jax
import jax.numpy as jnp
from jax import lax
from jax.experimental import pallas as pl
from jax.experimental.pallas import tpu as pltpu
from jax.experimental.pallas import tpu_sc as plsc

NROWS = 4096
NCOLS = 1001
ND = 1000
NNEG = 999
TOPK = 10
FMIN = float(jnp.finfo(jnp.float32).min)

NC, NS, L = 2, 16, 16
NW = NC * NS
RPW = NROWS // NW
RB = 16
NBLK = RPW // RB
NCHUNK = NNEG // L

LGBUF = RB * NCOLS + L
DMBUF = RB * ND + L


def _lanesum(v, lane):
    for sh in (8, 4, 2, 1):
        v = v + v.at[(lane + sh) & (L - 1)].get(mode="promise_in_bounds")
    return v[0]


def _sc_body(lg_hbm, dm_hbm, out_hbm, lg_v, dm_v, out_v):
    wid = lax.axis_index("s") * NC + lax.axis_index("c")
    row0 = wid * RPW
    lane = lax.iota(jnp.int32, L)
    tail_cmp = lane < (NNEG - NCHUNK * L)
    tail_dm = lane < (ND - NCHUNK * L)

    def block_body(blk, carry):
        hr_s, hr_c = carry
        r0 = row0 + blk * RB
        pltpu.sync_copy(lg_hbm.at[pl.ds(r0 * NCOLS, RB * NCOLS)],
                        lg_v.at[pl.ds(0, RB * NCOLS)])
        pltpu.sync_copy(dm_hbm.at[pl.ds(r0 * ND, RB * ND)],
                        dm_v.at[pl.ds(0, RB * ND)])

        def row_body(r, rcarry):
            hs, hc = rcarry
            bx = r * NCOLS
            bd = r * ND
            tvv = lg_v[pl.ds(bx + 1 + NNEG, L)]
            tdv = dm_v[pl.ds(bd + NNEG, L)]
            tv = tvv[0] + tdv[0].astype(jnp.float32) * FMIN
            cnt = jnp.zeros((L,), jnp.int32)
            dms = jnp.zeros((L,), jnp.int32)
            for k in range(NCHUNK):
                lgc = lg_v[pl.ds(bx + 1 + k * L, L)]
                dmc = dm_v[pl.ds(bd + k * L, L)]
                m = lgc + dmc.astype(jnp.float32) * FMIN
                cnt = cnt + jnp.where(m >= tv, 1, 0)
                dms = dms + dmc
            lgc = lg_v[pl.ds(bx + 1 + NCHUNK * L, L)]
            dmc = dm_v[pl.ds(bd + NCHUNK * L, L)]
            m = lgc + dmc.astype(jnp.float32) * FMIN
            cnt = cnt + jnp.where(tail_cmp & (m >= tv), 1, 0)
            dms = dms + jnp.where(tail_dm, dmc, 0)

            count = _lanesum(cnt, lane)
            w = jnp.where(_lanesum(dms, lane) != NNEG, 1.0, 0.0)
            hs = hs + jnp.where(count < TOPK, w, 0.0)
            hc = hc + w
            return hs, hc

        return lax.fori_loop(0, RB, row_body, (hr_s, hr_c))

    hr_s, hr_c = lax.fori_loop(0, NBLK, block_body, (0.0, 0.0))
    out_v[...] = jnp.where(lane == 0, hr_s, jnp.where(lane == 1, hr_c, 0.0))
    pltpu.sync_copy(out_v, out_hbm.at[wid])


_sc_metric = functools.partial(
    pl.kernel,
    out_type=jax.ShapeDtypeStruct((NW, L), jnp.float32),
    mesh=plsc.VectorSubcoreMesh(core_axis_name="c", subcore_axis_name="s"),
    scratch_types=[
        pltpu.VMEM((LGBUF,), jnp.float32),
        pltpu.VMEM((DMBUF,), jnp.int32),
        pltpu.VMEM((L,), jnp.float32),
    ],
)(_sc_body)


def kernel(logits, dup_mask):
    partials = _sc_metric(logits.reshape(-1), dup_mask.reshape(-1))
    hr_sum = jnp.sum(partials[:, 0])
    hr_count = jnp.sum(partials[:, 1])
    return logits, hr_sum, hr_count


# speedup vs baseline: 5.6266x; 1.1154x over previous
import functools

import jax
import jax.numpy as jnp
from jax import lax
from jax.experimental import pallas as pl
from jax.experimental.pallas import tpu as pltpu
from jax.experimental.pallas import tpu_sc as plsc

NROWS = 4096
NCOLS = 1001
ND = 1000
NNEG = 999
TOPK = 10
FMIN = float(jnp.finfo(jnp.float32).min)

NC, NS, L = 2, 16, 16
NW = NC * NS
RPW = NROWS // NW
RB = 16
NBLK = RPW // RB
NCHUNK = NNEG // L

LGBUF = RB * NCOLS + L
DMBUF = RB * ND + L

PACK = 12


def _sc_body(lg_hbm, dm_hbm, out_hbm,
             lg_v0, dm_v0, lg_v1, dm_v1, out_v,
             s_lg0, s_dm0, s_lg1, s_dm1):
    wid = lax.axis_index("s") * NC + lax.axis_index("c")
    row0 = wid * RPW
    lane = lax.iota(jnp.int32, L)
    tail_cmp = lane < (NNEG - NCHUNK * L)
    tail_dm = lane < (ND - NCHUNK * L)

    lg_bufs = (lg_v0, lg_v1)
    dm_bufs = (dm_v0, dm_v1)
    lg_sems = (s_lg0, s_lg1)
    dm_sems = (s_dm0, s_dm1)

    def start(blk):
        b = blk % 2
        r0 = row0 + blk * RB
        h_lg = pltpu.make_async_copy(
            lg_hbm.at[pl.ds(r0 * NCOLS, RB * NCOLS)],
            lg_bufs[b].at[pl.ds(0, RB * NCOLS)], lg_sems[b])
        h_dm = pltpu.make_async_copy(
            dm_hbm.at[pl.ds(r0 * ND, RB * ND)],
            dm_bufs[b].at[pl.ds(0, RB * ND)], dm_sems[b])
        h_lg.start()
        h_dm.start()
        return h_lg, h_dm

    def row_body(lg_v, dm_v):
        def body(r, rcarry):
            hs, hc = rcarry
            bx = r * NCOLS
            bd = r * ND
            tvv = lg_v[pl.ds(bx + 1 + NNEG, L)]
            tdv = dm_v[pl.ds(bd + NNEG, L)]
            tv = tvv[0] + tdv[0].astype(jnp.float32) * FMIN
            cnt = jnp.zeros((L,), jnp.int32)
            dms = jnp.zeros((L,), jnp.int32)
            for k in range(NCHUNK):
                lgc = lg_v[pl.ds(bx + 1 + k * L, L)]
                dmc = dm_v[pl.ds(bd + k * L, L)]
                m = lgc + dmc.astype(jnp.float32) * FMIN
                cnt = cnt + jnp.where(m >= tv, 1, 0)
                dms = dms + dmc
            lgc = lg_v[pl.ds(bx + 1 + NCHUNK * L, L)]
            dmc = dm_v[pl.ds(bd + NCHUNK * L, L)]
            m = lgc + dmc.astype(jnp.float32) * FMIN
            cnt = cnt + jnp.where(tail_cmp & (m >= tv), 1, 0)
            dms = dms + jnp.where(tail_dm, dmc, 0)

            packed = cnt + (dms << PACK)
            for sh in (8, 4, 2, 1):
                packed = packed + packed.at[(lane + sh) & (L - 1)].get(
                    mode="promise_in_bounds")
            s = packed[0]
            count = s & ((1 << PACK) - 1)
            w = jnp.where((s >> PACK) != NNEG, 1.0, 0.0)
            hs = hs + jnp.where(count < TOPK, w, 0.0)
            hc = hc + w
            return hs, hc
        return body

    handles = start(0)
    hr = (0.0, 0.0)
    for blk in range(NBLK):
        nxt = start(blk + 1) if blk + 1 < NBLK else None
        handles[0].wait()
        handles[1].wait()
        b = blk % 2
        hr = lax.fori_loop(0, RB, row_body(lg_bufs[b], dm_bufs[b]), hr)
        handles = nxt

    hr_s, hr_c = hr
    out_v[...] = jnp.where(lane == 0, hr_s, jnp.where(lane == 1, hr_c, 0.0))
    pltpu.sync_copy(out_v, out_hbm.at[wid])


_sc_metric = functools.partial(
    pl.kernel,
    out_type=jax.ShapeDtypeStruct((NW, L), jnp.float32),
    mesh=plsc.VectorSubcoreMesh(core_axis_name="c", subcore_axis_name="s"),
    scratch_types=[
        pltpu.VMEM((LGBUF,), jnp.float32),
        pltpu.VMEM((DMBUF,), jnp.int32),
        pltpu.VMEM((LGBUF,), jnp.float32),
        pltpu.VMEM((DMBUF,), jnp.int32),
        pltpu.VMEM((L,), jnp.float32),
        pltpu.SemaphoreType.DMA,
        pltpu.SemaphoreType.DMA,
        pltpu.SemaphoreType.DMA,
        pltpu.SemaphoreType.DMA,
    ],
)(_sc_body)


def kernel(logits, dup_mask):
    partials = _sc_metric(logits.reshape(-1), dup_mask.reshape(-1))
    hr_sum = jnp.sum(partials[:, 0])
    hr_count = jnp.sum(partials[:, 1])
    return logits, hr_sum, hr_count


# speedup vs baseline: 7.5194x; 1.3364x over previous
import functools

import jax
import jax.numpy as jnp
from jax import lax
from jax.experimental import pallas as pl
from jax.experimental.pallas import tpu as pltpu
from jax.experimental.pallas import tpu_sc as plsc

NROWS = 4096
NCOLS = 1001
ND = 1000
NNEG = 999
TOPK = 10
FMIN = float(jnp.finfo(jnp.float32).min)

NC, NS, L = 2, 16, 16
NW = NC * NS
RPW = NROWS // NW
RB = 16
NBLK = RPW // RB
NCHUNK = NNEG // L

LGBUF = RB * NCOLS + L
DMBUF = RB * ND + L

PACK = 12


def _sc_body(lg_hbm, dm_hbm, out_hbm,
             lg_v0, dm_v0, lg_v1, dm_v1, out_v,
             s_lg0, s_dm0, s_lg1, s_dm1):
    wid = lax.axis_index("s") * NC + lax.axis_index("c")
    row0 = wid * RPW
    lane = lax.iota(jnp.int32, L)
    tail_cmp = lane < (NNEG - NCHUNK * L)
    tail_dm = lane < (ND - NCHUNK * L)

    lg_bufs = (lg_v0, lg_v1)
    dm_bufs = (dm_v0, dm_v1)
    lg_sems = (s_lg0, s_lg1)
    dm_sems = (s_dm0, s_dm1)

    def start(blk):
        b = blk % 2
        r0 = row0 + blk * RB
        h_lg = pltpu.make_async_copy(
            lg_hbm.at[pl.ds(r0 * NCOLS, RB * NCOLS)],
            lg_bufs[b].at[pl.ds(0, RB * NCOLS)], lg_sems[b])
        h_dm = pltpu.make_async_copy(
            dm_hbm.at[pl.ds(r0 * ND, RB * ND)],
            dm_bufs[b].at[pl.ds(0, RB * ND)], dm_sems[b])
        h_lg.start()
        h_dm.start()
        return h_lg, h_dm

    def row_body(lg_v, dm_v):
        def body(r, rcarry):
            hs, hc = rcarry
            bx = r * NCOLS
            bd = r * ND
            tvv = lg_v[pl.ds(bx + 1 + NNEG, L)]
            tdv = dm_v[pl.ds(bd + NNEG, L)]
            tv = tvv[0] + tdv[0].astype(jnp.float32) * FMIN
            def chunk(k, ccarry):
                cnt, dms = ccarry
                lgc = lg_v[pl.ds(bx + 1 + k * L, L)]
                dmc = dm_v[pl.ds(bd + k * L, L)]
                m = lgc + dmc.astype(jnp.float32) * FMIN
                cnt = cnt + jnp.where(m >= tv, 1, 0)
                dms = dms + dmc
                return cnt, dms

            cnt, dms = lax.fori_loop(
                0, NCHUNK, chunk,
                (jnp.zeros((L,), jnp.int32), jnp.zeros((L,), jnp.int32)),
                unroll=2)
            lgc = lg_v[pl.ds(bx + 1 + NCHUNK * L, L)]
            dmc = dm_v[pl.ds(bd + NCHUNK * L, L)]
            m = lgc + dmc.astype(jnp.float32) * FMIN
            cnt = cnt + jnp.where(tail_cmp & (m >= tv), 1, 0)
            dms = dms + jnp.where(tail_dm, dmc, 0)

            packed = cnt + (dms << PACK)
            for sh in (8, 4, 2, 1):
                packed = packed + packed.at[(lane + sh) & (L - 1)].get(
                    mode="promise_in_bounds")
            s = packed[0]
            count = s & ((1 << PACK) - 1)
            w = jnp.where((s >> PACK) != NNEG, 1.0, 0.0)
            hs = hs + jnp.where(count < TOPK, w, 0.0)
            hc = hc + w
            return hs, hc
        return body

    handles = start(0)
    hr = (0.0, 0.0)
    for blk in range(NBLK):
        nxt = start(blk + 1) if blk + 1 < NBLK else None
        handles[0].wait()
        handles[1].wait()
        b = blk % 2
        hr = lax.fori_loop(0, RB, row_body(lg_bufs[b], dm_bufs[b]), hr,
                           unroll=1)
        handles = nxt

    hr_s, hr_c = hr
    out_v[...] = jnp.where(lane == 0, hr_s, jnp.where(lane == 1, hr_c, 0.0))
    pltpu.sync_copy(out_v, out_hbm.at[wid])


_sc_metric = functools.partial(
    pl.kernel,
    out_type=jax.ShapeDtypeStruct((NW, L), jnp.float32),
    mesh=plsc.VectorSubcoreMesh(core_axis_name="c", subcore_axis_name="s"),
    scratch_types=[
        pltpu.VMEM((LGBUF,), jnp.float32),
        pltpu.VMEM((DMBUF,), jnp.int32),
        pltpu.VMEM((LGBUF,), jnp.float32),
        pltpu.VMEM((DMBUF,), jnp.int32),
        pltpu.VMEM((L,), jnp.float32),
        pltpu.SemaphoreType.DMA,
        pltpu.SemaphoreType.DMA,
        pltpu.SemaphoreType.DMA,
        pltpu.SemaphoreType.DMA,
    ],
)(_sc_body)


def kernel(logits, dup_mask):
    partials = _sc_metric(logits.reshape(-1), dup_mask.reshape(-1))
    hr_sum = jnp.sum(partials[:, 0])
    hr_count = jnp.sum(partials[:, 1])
    return logits, hr_sum, hr_count


# speedup vs baseline: 7.7632x; 1.0324x over previous
import functools

import jax
import jax.numpy as jnp
from jax import lax
from jax.experimental import pallas as pl
from jax.experimental.pallas import tpu as pltpu
from jax.experimental.pallas import tpu_sc as plsc

NROWS = 4096
NCOLS = 1001
ND = 1000
NNEG = 999
TOPK = 10
FMIN = float(jnp.finfo(jnp.float32).min)

NC, NS, L = 2, 16, 16
NW = NC * NS
RPW = NROWS // NW
RB = 16
NBLK = RPW // RB
NCHUNK = NNEG // L

LGBUF = RB * NCOLS + L
DMBUF = RB * ND + L

PACK = 12


def _sc_body(lg_hbm, dm_hbm, out_hbm,
             lg_v0, dm_v0, lg_v1, dm_v1, out_v,
             s_lg0, s_dm0, s_lg1, s_dm1):
    wid = lax.axis_index("s") * NC + lax.axis_index("c")
    row0 = wid * RPW
    lane = lax.iota(jnp.int32, L)
    tail_cmp = lane < (NNEG - NCHUNK * L)
    tail_dm = lane < (ND - NCHUNK * L)

    lg_bufs = (lg_v0, lg_v1)
    dm_bufs = (dm_v0, dm_v1)
    lg_sems = (s_lg0, s_lg1)
    dm_sems = (s_dm0, s_dm1)

    def start(blk):
        b = blk % 2
        r0 = row0 + blk * RB
        h_lg = pltpu.make_async_copy(
            lg_hbm.at[pl.ds(r0 * NCOLS, RB * NCOLS)],
            lg_bufs[b].at[pl.ds(0, RB * NCOLS)], lg_sems[b])
        h_dm = pltpu.make_async_copy(
            dm_hbm.at[pl.ds(r0 * ND, RB * ND)],
            dm_bufs[b].at[pl.ds(0, RB * ND)], dm_sems[b])
        h_lg.start()
        h_dm.start()
        return h_lg, h_dm

    def row_body(lg_v, dm_v):
        def body(r, rcarry):
            hs, hc = rcarry
            bx = r * NCOLS
            bd = r * ND
            tvv = lg_v[pl.ds(bx + 1 + NNEG, L)]
            tdv = dm_v[pl.ds(bd + NNEG, L)]
            tv = tvv[0] + tdv[0].astype(jnp.float32) * FMIN
            def chunk(k, ccarry):
                cnt, dms = ccarry
                lgc = lg_v[pl.ds(bx + 1 + k * L, L)]
                dmc = dm_v[pl.ds(bd + k * L, L)]
                m = lgc + dmc.astype(jnp.float32) * FMIN
                cnt = cnt + jnp.where(m >= tv, 1, 0)
                dms = dms + dmc
                return cnt, dms

            cnt, dms = lax.fori_loop(
                0, NCHUNK, chunk,
                (jnp.zeros((L,), jnp.int32), jnp.zeros((L,), jnp.int32)),
                unroll=4)
            lgc = lg_v[pl.ds(bx + 1 + NCHUNK * L, L)]
            dmc = dm_v[pl.ds(bd + NCHUNK * L, L)]
            m = lgc + dmc.astype(jnp.float32) * FMIN
            cnt = cnt + jnp.where(tail_cmp & (m >= tv), 1, 0)
            dms = dms + jnp.where(tail_dm, dmc, 0)

            packed = cnt + (dms << PACK)
            for sh in (8, 4, 2, 1):
                packed = packed + packed.at[(lane + sh) & (L - 1)].get(
                    mode="promise_in_bounds")
            s = packed[0]
            count = s & ((1 << PACK) - 1)
            w = jnp.where((s >> PACK) != NNEG, 1.0, 0.0)
            hs = hs + jnp.where(count < TOPK, w, 0.0)
            hc = hc + w
            return hs, hc
        return body

    handles = start(0)
    hr = (0.0, 0.0)
    for blk in range(NBLK):
        nxt = start(blk + 1) if blk + 1 < NBLK else None
        handles[0].wait()
        handles[1].wait()
        b = blk % 2
        hr = lax.fori_loop(0, RB, row_body(lg_bufs[b], dm_bufs[b]), hr,
                           unroll=1)
        handles = nxt

    hr_s, hr_c = hr
    out_v[...] = jnp.where(lane == 0, hr_s, jnp.where(lane == 1, hr_c, 0.0))
    pltpu.sync_copy(out_v, out_hbm.at[wid])


_sc_metric = functools.partial(
    pl.kernel,
    out_type=jax.ShapeDtypeStruct((NW, L), jnp.float32),
    mesh=plsc.VectorSubcoreMesh(core_axis_name="c", subcore_axis_name="s"),
    scratch_types=[
        pltpu.VMEM((LGBUF,), jnp.float32),
        pltpu.VMEM((DMBUF,), jnp.int32),
        pltpu.VMEM((LGBUF,), jnp.float32),
        pltpu.VMEM((DMBUF,), jnp.int32),
        pltpu.VMEM((L,), jnp.float32),
        pltpu.SemaphoreType.DMA,
        pltpu.SemaphoreType.DMA,
        pltpu.SemaphoreType.DMA,
        pltpu.SemaphoreType.DMA,
    ],
)(_sc_body)


def kernel(logits, dup_mask):
    partials = _sc_metric(logits.reshape(-1), dup_mask.reshape(-1))
    hr_sum = jnp.sum(partials[:, 0])
    hr_count = jnp.sum(partials[:, 1])
    return logits, hr_sum, hr_count


# speedup vs baseline: 10.2277x; 1.3175x over previous
import functools

import jax
import jax.numpy as jnp
from jax import lax
from jax.experimental import pallas as pl
from jax.experimental.pallas import tpu as pltpu
from jax.experimental.pallas import tpu_sc as plsc

NROWS = 4096
NCOLS = 1001
ND = 1000
NNEG = 999
TOPK = 10
FMIN = float(jnp.finfo(jnp.float32).min)

NC, NS, L = 2, 16, 16
NW = NC * NS
RPW = NROWS // NW
RB = 16
NBLK = RPW // RB

PACK = 12


def _sc_body(lg_hbm, dm_hbm, out_hbm,
             lg_v0, dm_v0, lg_v1, dm_v1, out_v,
             s_lg0, s_dm0, s_lg1, s_dm1):
    wid = lax.axis_index("s") * NC + lax.axis_index("c")
    row0 = wid * RPW
    lane = lax.iota(jnp.int32, L)
    rot1 = (lane + 1) & (L - 1)

    lg_bufs = (lg_v0, lg_v1)
    dm_bufs = (dm_v0, dm_v1)
    lg_sems = (s_lg0, s_lg1)
    dm_sems = (s_dm0, s_dm1)

    def start(blk):
        b = blk % 2
        r0 = row0 + blk * RB
        h_lg = pltpu.make_async_copy(
            lg_hbm.at[pl.ds(r0, RB)], lg_bufs[b], lg_sems[b])
        h_dm = pltpu.make_async_copy(
            dm_hbm.at[pl.ds(r0, RB)], dm_bufs[b], dm_sems[b])
        h_lg.start()
        h_dm.start()
        return h_lg, h_dm

    def row_body(lg_v, dm_v):
        def body(r, rcarry):
            hs, hc = rcarry
            lgE2 = lg_v[r, pl.ds(NCOLS - L, L)]
            dmE2 = dm_v[r, pl.ds(ND - L, L)]
            tv = lgE2[L - 1] + dmE2[L - 1].astype(jnp.float32) * FMIN

            def chunk(j, ccarry):
                cnt, dms, rot = ccarry
                a1 = lg_v[r, pl.ds(16 * j + 16, L)]
                rot1v = a1.at[rot1].get(mode="promise_in_bounds")
                merged = jnp.where(lane < L - 1, rot, rot1v)
                dmc = dm_v[r, pl.ds(16 * j, L)]
                m = merged + dmc.astype(jnp.float32) * FMIN
                cnt = cnt + jnp.where(m >= tv, 1, 0)
                dms = dms + dmc
                return cnt, dms, rot1v

            a0 = lg_v[r, pl.ds(0, L)]
            cnt, dms, _ = lax.fori_loop(
                0, 61, chunk,
                (jnp.zeros((L,), jnp.int32), jnp.zeros((L,), jnp.int32),
                 a0.at[rot1].get(mode="promise_in_bounds")),
                unroll=4)
            lgE1 = lg_v[r, pl.ds(969, L)]
            dmE1 = dm_v[r, pl.ds(968, L)]
            hi8 = lane >= 8
            m = lgE1 + dmE1.astype(jnp.float32) * FMIN
            cnt = cnt + jnp.where(hi8 & (m >= tv), 1, 0)
            dms = dms + jnp.where(hi8, dmE1, 0)
            m = lgE2 + dmE2.astype(jnp.float32) * FMIN
            cnt = cnt + jnp.where((lane < L - 1) & (m >= tv), 1, 0)
            dms = dms + dmE2

            packed = cnt + (dms << PACK)
            for sh in (8, 4, 2, 1):
                packed = packed + packed.at[(lane + sh) & (L - 1)].get(
                    mode="promise_in_bounds")
            s = packed[0]
            count = s & ((1 << PACK) - 1)
            w = jnp.where((s >> PACK) != NNEG, 1.0, 0.0)
            hs = hs + jnp.where(count < TOPK, w, 0.0)
            hc = hc + w
            return hs, hc
        return body

    handles = start(0)
    hr = (0.0, 0.0)
    for blk in range(NBLK):
        nxt = start(blk + 1) if blk + 1 < NBLK else None
        handles[0].wait()
        handles[1].wait()
        b = blk % 2
        hr = lax.fori_loop(0, RB, row_body(lg_bufs[b], dm_bufs[b]), hr,
                           unroll=1)
        handles = nxt

    hr_s, hr_c = hr
    out_v[...] = jnp.where(lane == 0, hr_s, jnp.where(lane == 1, hr_c, 0.0))
    pltpu.sync_copy(out_v, out_hbm.at[wid])


_sc_metric = functools.partial(
    pl.kernel,
    out_type=jax.ShapeDtypeStruct((NW, L), jnp.float32),
    mesh=plsc.VectorSubcoreMesh(core_axis_name="c", subcore_axis_name="s"),
    scratch_types=[
        pltpu.VMEM((RB, NCOLS), jnp.float32),
        pltpu.VMEM((RB, ND), jnp.int32),
        pltpu.VMEM((RB, NCOLS), jnp.float32),
        pltpu.VMEM((RB, ND), jnp.int32),
        pltpu.VMEM((L,), jnp.float32),
        pltpu.SemaphoreType.DMA,
        pltpu.SemaphoreType.DMA,
        pltpu.SemaphoreType.DMA,
        pltpu.SemaphoreType.DMA,
    ],
    compiler_params=pltpu.CompilerParams(use_tc_tiling_on_sc=True, needs_layout_passes=False),
)(_sc_body)


def kernel(logits, dup_mask):
    partials = _sc_metric(logits, dup_mask)
    hr_sum = jnp.sum(partials[:, 0])
    hr_count = jnp.sum(partials[:, 1])
    return logits, hr_sum, hr_count


# speedup vs baseline: 16.2777x; 1.5915x over previous
import functools

import jax
import jax.numpy as jnp
from jax import lax
from jax.experimental import pallas as pl
from jax.experimental.pallas import tpu as pltpu
from jax.experimental.pallas import tpu_sc as plsc

NROWS = 4096
NCOLS = 1001
ND = 1000
NNEG = 999
TOPK = 10
FMIN = float(jnp.finfo(jnp.float32).min)

NC, NS, L = 2, 16, 16
NW = NC * NS
RPW = NROWS // NW
NG = RPW // L

CC = 192
CHUNKS = [
    (0,   CC, CC, 0,   191),
    (184, CC, CC, 191, 375),
    (368, CC, CC, 375, 559),
    (552, CC, CC, 559, 743),
    (736, CC, CC, 743, 927),
    (920, 81, 80, 927, 999),
]

PACK = 12


def _sc_body(lg_hbm, dm_hbm, out_hbm,
             lg_v0, dm_v0, lg_v1, dm_v1, out_v,
             s_lg0, s_dm0, s_lg1, s_dm1):
    wid = lax.axis_index("s") * NC + lax.axis_index("c")
    r0 = wid * RPW
    lane = lax.iota(jnp.int32, L)

    lg_bufs = (lg_v0, lg_v1)
    dm_bufs = (dm_v0, dm_v1)
    lg_sems = (s_lg0, s_lg1)
    dm_sems = (s_dm0, s_dm1)

    ORDER = [5, 0, 1, 2, 3, 4]

    def start(pos):
        c0, cl, cd, _, _ = CHUNKS[ORDER[pos]]
        b = pos % 2
        h_lg = pltpu.make_async_copy(
            lg_hbm.at[pl.ds(c0, cl), pl.ds(r0, RPW)],
            lg_bufs[b].at[pl.ds(0, cl)], lg_sems[b])
        h_dm = pltpu.make_async_copy(
            dm_hbm.at[pl.ds(c0, cd), pl.ds(r0, RPW)],
            dm_bufs[b].at[pl.ds(0, cd)], dm_sems[b])
        h_lg.start()
        h_dm.start()
        return h_lg, h_dm

    accs = [None] * NG
    tvs = [None] * NG
    handles = start(0)
    for pos, ci in enumerate(ORDER):
        c0, _, _, jlo, jhi = CHUNKS[ci]
        nxt = start(pos + 1) if pos + 1 < len(ORDER) else None
        handles[0].wait()
        handles[1].wait()
        lg_v = lg_bufs[pos % 2]
        dm_v = dm_bufs[pos % 2]
        for g in range(NG):
            if pos == 0:
                tvd = dm_v[ND - 1 - c0, pl.ds(g * L, L)]
                tvs[g] = (lg_v[NCOLS - 1 - c0, pl.ds(g * L, L)]
                          + tvd.astype(jnp.float32) * FMIN)
                accs[g] = (jnp.zeros((L,), jnp.int32), tvd)
            tv = tvs[g]

            def jbody(jl, ccarry, lg_v=lg_v, dm_v=dm_v, g=g, tv=tv):
                cnt, dms = ccarry
                lgc = lg_v[jl + 1, pl.ds(g * L, L)]
                dmc = dm_v[jl, pl.ds(g * L, L)]
                m = lgc + dmc.astype(jnp.float32) * FMIN
                cnt = cnt + jnp.where(m >= tv, 1, 0)
                dms = dms + dmc
                return cnt, dms

            accs[g] = lax.fori_loop(jlo - c0, jhi - c0, jbody, accs[g],
                                    unroll=4)
        handles = nxt

    hrp = jnp.zeros((L,), jnp.int32)
    for g in range(NG):
        cnt, dtot = accs[g]
        wi = jnp.where(dtot != NNEG, 1, 0)
        hit = jnp.where((cnt < TOPK) & (dtot != NNEG), 1, 0)
        hrp = hrp + wi + (hit << PACK)
    for sh in (8, 4, 2, 1):
        hrp = hrp + hrp.at[(lane + sh) & (L - 1)].get(
            mode="promise_in_bounds")
    s = hrp[0]
    hr_s = (s >> PACK).astype(jnp.float32)
    hr_c = (s & ((1 << PACK) - 1)).astype(jnp.float32)
    out_v[...] = jnp.where(lane == 0, hr_s, jnp.where(lane == 1, hr_c, 0.0))
    pltpu.sync_copy(out_v, out_hbm.at[wid])


_sc_metric = functools.partial(
    pl.kernel,
    out_type=jax.ShapeDtypeStruct((NW, L), jnp.float32),
    mesh=plsc.VectorSubcoreMesh(core_axis_name="c", subcore_axis_name="s"),
    scratch_types=[
        pltpu.VMEM((CC, RPW), jnp.float32),
        pltpu.VMEM((CC, RPW), jnp.int32),
        pltpu.VMEM((CC, RPW), jnp.float32),
        pltpu.VMEM((CC, RPW), jnp.int32),
        pltpu.VMEM((L,), jnp.float32),
        pltpu.SemaphoreType.DMA,
        pltpu.SemaphoreType.DMA,
        pltpu.SemaphoreType.DMA,
        pltpu.SemaphoreType.DMA,
    ],
    compiler_params=pltpu.CompilerParams(use_tc_tiling_on_sc=True,
                                         needs_layout_passes=False),
)(_sc_body)


def kernel(logits, dup_mask):
    partials = _sc_metric(logits.T, dup_mask.T)
    hr_sum = jnp.sum(partials[:, 0])
    hr_count = jnp.sum(partials[:, 1])
    return logits, hr_sum, hr_count
